# Initial kernel scaffold; baseline (speedup 1.0000x reference)
import jax
import jax.numpy as jnp
from jax.experimental import pallas as pl


def kernel(input_ids, token_type_ids, word_table, pos_table, type_table, gamma, beta):
    raise NotImplementedError("write your pallas kernel here")


# speedup vs baseline: 1.8645x; 1.8645x over previous
import functools

import jax
import jax.numpy as jnp
from jax.experimental import pallas as pl
from jax.experimental.pallas import tpu as pltpu
from jax.experimental.pallas import tpu_sc as plsc

_EPS = 1e-12
_GATHER_W = 128
_BB = 8


def _sc_gather(word_table, idx_flat):
    n = idx_flat.shape[0]
    h = word_table.shape[1]
    mesh = plsc.VectorSubcoreMesh(core_axis_name="c", subcore_axis_name="s")
    idx2 = idx_flat.reshape(1, n)

    @functools.partial(
        pl.kernel,
        out_type=jax.ShapeDtypeStruct((n, h), jnp.float32),
        mesh=mesh,
        compiler_params=pltpu.CompilerParams(use_tc_tiling_on_sc=False),
    )
    def gather_kernel(table_hbm, i_hbm, o_hbm):
        def body(i_vmem, o_vmem):
            pltpu.sync_copy(table_hbm.at[i_vmem.at[0]], o_vmem)

        pltpu.emit_pipeline(
            body,
            grid=(n // _GATHER_W,),
            in_specs=[pl.BlockSpec((1, _GATHER_W), lambda i: (0, i))],
            out_specs=[pl.BlockSpec((_GATHER_W, h), lambda i: (i, 0))],
            core_axis_name=("c", "s"),
            dimension_semantics=(pltpu.PARALLEL,),
        )(i_hbm, o_hbm)

    return gather_kernel(word_table, idx2)


def _ln_body(wemb_ref, tt_ref, pos_ref, ttab_ref, gamma_ref, beta_ref, out_ref):
    emb = wemb_ref[...]
    tt = tt_ref[...].astype(jnp.float32)
    pos = pos_ref[...]
    ttab = ttab_ref[...]
    t0 = ttab[0:1, :]
    dt = ttab[1:2, :] - t0
    type_emb = t0[None] + tt[..., None] * dt[None]
    emb = emb + pos[None] + type_emb
    mean = jnp.mean(emb, axis=-1, keepdims=True)
    cen = emb - mean
    var = jnp.mean(cen * cen, axis=-1, keepdims=True)
    normed = cen * jax.lax.rsqrt(var + _EPS)
    gamma = gamma_ref[...][0]
    beta = beta_ref[...][0]
    out_ref[...] = normed * gamma + beta


def _tc_layernorm(wemb, token_type_ids, pos_s, type_table, gamma, beta):
    b, s, h = wemb.shape
    ttab = jnp.pad(type_table, ((0, 6), (0, 0)))
    gamma8 = jnp.pad(gamma.reshape(1, h), ((0, 7), (0, 0)))
    beta8 = jnp.pad(beta.reshape(1, h), ((0, 7), (0, 0)))
    grid = (b // _BB,)
    return pl.pallas_call(
        _ln_body,
        grid=grid,
        in_specs=[
            pl.BlockSpec((_BB, s, h), lambda i: (i, 0, 0)),
            pl.BlockSpec((_BB, s), lambda i: (i, 0)),
            pl.BlockSpec((s, h), lambda i: (0, 0)),
            pl.BlockSpec((8, h), lambda i: (0, 0)),
            pl.BlockSpec((8, h), lambda i: (0, 0)),
            pl.BlockSpec((8, h), lambda i: (0, 0)),
        ],
        out_specs=pl.BlockSpec((_BB, s, h), lambda i: (i, 0, 0)),
        out_shape=jax.ShapeDtypeStruct((b, s, h), jnp.float32),
    )(wemb, token_type_ids, pos_s, ttab, gamma8, beta8)


def kernel(input_ids, token_type_ids, word_table, pos_table, type_table, gamma, beta):
    b, s = input_ids.shape
    h = word_table.shape[1]
    idx_flat = input_ids.reshape(-1)
    wemb = _sc_gather(word_table, idx_flat).reshape(b, s, h)
    pos_s = pos_table[:s]
    return _tc_layernorm(wemb, token_type_ids, pos_s, type_table, gamma, beta)


# speedup vs baseline: 1.9308x; 1.0355x over previous
import functools

import jax
import jax.numpy as jnp
from jax.experimental import pallas as pl
from jax.experimental.pallas import tpu as pltpu
from jax.experimental.pallas import tpu_sc as plsc

_EPS = 1e-12
_GATHER_W = 128
_BP = 4


def _sc_gather(word_table, idx_flat):
    n = idx_flat.shape[0]
    h = word_table.shape[1]
    mesh = plsc.VectorSubcoreMesh(core_axis_name="c", subcore_axis_name="s")
    idx2 = idx_flat.reshape(1, n)

    @functools.partial(
        pl.kernel,
        out_type=jax.ShapeDtypeStruct((n, h), jnp.float32),
        mesh=mesh,
        compiler_params=pltpu.CompilerParams(use_tc_tiling_on_sc=False),
    )
    def gather_kernel(table_hbm, i_hbm, o_hbm):
        def body(i_vmem, o_vmem):
            pltpu.sync_copy(table_hbm.at[i_vmem.at[0]], o_vmem)

        pltpu.emit_pipeline(
            body,
            grid=(n // _GATHER_W,),
            in_specs=[pl.BlockSpec((1, _GATHER_W), lambda i: (0, i))],
            out_specs=[pl.BlockSpec((_GATHER_W, h), lambda i: (i, 0))],
            core_axis_name=("c", "s"),
            dimension_semantics=(pltpu.PARALLEL,),
        )(i_hbm, o_hbm)

    return gather_kernel(word_table, idx2)


def _ln_half(emb, tt, pos, t0, dt, gamma, beta):
    type_emb = t0[None] + tt[..., None] * dt[None]
    emb = emb + pos[None] + type_emb
    mean = jnp.mean(emb, axis=-1, keepdims=True)
    cen = emb - mean
    var = jnp.mean(cen * cen, axis=-1, keepdims=True)
    return cen * jax.lax.rsqrt(var + _EPS) * gamma + beta


def _ln_body(wemb2_ref, tt_ref, pos_ref, ttab_ref, gamma_ref, beta_ref, out_ref):
    h = out_ref.shape[2]
    w2 = wemb2_ref[...]
    xa = w2[:, :, :h]
    xb = w2[:, :, h:]
    tt = tt_ref[...].astype(jnp.float32)
    pos = pos_ref[...]
    ttab = ttab_ref[...]
    t0 = ttab[0:1, :]
    dt = ttab[1:2, :] - t0
    gamma = gamma_ref[...][0]
    beta = beta_ref[...][0]
    ya = _ln_half(xa, tt[:, 0, :], pos, t0, dt, gamma, beta)
    yb = _ln_half(xb, tt[:, 1, :], pos, t0, dt, gamma, beta)
    for p in range(ya.shape[0]):
        out_ref[2 * p] = ya[p]
        out_ref[2 * p + 1] = yb[p]


def _tc_layernorm(wemb2, tt_pairs, pos_s, type_table, gamma, beta):
    bp2, s, h2 = wemb2.shape
    h = h2 // 2
    ttab = jnp.pad(type_table, ((0, 6), (0, 0)))
    gamma8 = jnp.pad(gamma.reshape(1, h), ((0, 7), (0, 0)))
    beta8 = jnp.pad(beta.reshape(1, h), ((0, 7), (0, 0)))
    grid = (bp2 // _BP,)
    return pl.pallas_call(
        _ln_body,
        grid=grid,
        in_specs=[
            pl.BlockSpec((_BP, s, h2), lambda i: (i, 0, 0)),
            pl.BlockSpec((_BP, 2, s), lambda i: (i, 0, 0)),
            pl.BlockSpec((s, h), lambda i: (0, 0)),
            pl.BlockSpec((8, h), lambda i: (0, 0)),
            pl.BlockSpec((8, h), lambda i: (0, 0)),
            pl.BlockSpec((8, h), lambda i: (0, 0)),
        ],
        out_specs=pl.BlockSpec((2 * _BP, s, h), lambda i: (i, 0, 0)),
        out_shape=jax.ShapeDtypeStruct((2 * bp2, s, h), jnp.float32),
    )(wemb2, tt_pairs, pos_s, ttab, gamma8, beta8)


def kernel(input_ids, token_type_ids, word_table, pos_table, type_table, gamma, beta):
    b, s = input_ids.shape
    h = word_table.shape[1]
    ids_perm = jnp.stack([input_ids[0::2], input_ids[1::2]], axis=-1)
    idx_flat = ids_perm.reshape(-1)
    wemb2 = _sc_gather(word_table, idx_flat).reshape(b // 2, s, 2 * h)
    tt_pairs = jnp.stack([token_type_ids[0::2], token_type_ids[1::2]], axis=1)
    pos_s = pos_table[:s]
    return _tc_layernorm(wemb2, tt_pairs, pos_s, type_table, gamma, beta)


# speedup vs baseline: 2.0989x; 1.0871x over previous
import functools

import jax
import jax.numpy as jnp
from jax.experimental import pallas as pl
from jax.experimental.pallas import tpu as pltpu
from jax.experimental.pallas import tpu_sc as plsc

_EPS = 1e-12
_GATHER_W = 128
_BB = 8


def _sc_gather(table_pad, idx_flat):
    n = idx_flat.shape[0]
    h2 = table_pad.shape[1]
    mesh = plsc.VectorSubcoreMesh(core_axis_name="c", subcore_axis_name="s")
    idx2 = idx_flat.reshape(1, n)

    @functools.partial(
        pl.kernel,
        out_type=jax.ShapeDtypeStruct((n, h2), jnp.float32),
        mesh=mesh,
        compiler_params=pltpu.CompilerParams(use_tc_tiling_on_sc=False),
    )
    def gather_kernel(table_hbm, i_hbm, o_hbm):
        def body(i_vmem, o_vmem):
            pltpu.sync_copy(table_hbm.at[i_vmem.at[0]], o_vmem)

        pltpu.emit_pipeline(
            body,
            grid=(n // _GATHER_W,),
            in_specs=[pl.BlockSpec((1, _GATHER_W), lambda i: (0, i))],
            out_specs=[pl.BlockSpec((_GATHER_W, h2), lambda i: (i, 0))],
            core_axis_name=("c", "s"),
            dimension_semantics=(pltpu.PARALLEL,),
        )(i_hbm, o_hbm)

    return gather_kernel(table_pad, idx2)


def _ln_body(wemb_ref, tt_ref, pos_ref, ttab_ref, gamma_ref, beta_ref, out_ref):
    h = out_ref.shape[2]
    emb = wemb_ref[:, :, :h]
    tt = tt_ref[...].astype(jnp.float32)
    pos = pos_ref[...]
    ttab = ttab_ref[...]
    t0 = ttab[0:1, :]
    dt = ttab[1:2, :] - t0
    type_emb = t0[None] + tt[..., None] * dt[None]
    emb = emb + pos[None] + type_emb
    mean = jnp.mean(emb, axis=-1, keepdims=True)
    cen = emb - mean
    var = jnp.mean(cen * cen, axis=-1, keepdims=True)
    normed = cen * jax.lax.rsqrt(var + _EPS)
    gamma = gamma_ref[...][0]
    beta = beta_ref[...][0]
    out_ref[...] = normed * gamma + beta


def _tc_layernorm(wemb, token_type_ids, pos_s, type_table, gamma, beta):
    b, s, h2 = wemb.shape
    h = h2 // 2
    ttab = jnp.pad(type_table, ((0, 6), (0, 0)))
    gamma8 = jnp.pad(gamma.reshape(1, h), ((0, 7), (0, 0)))
    beta8 = jnp.pad(beta.reshape(1, h), ((0, 7), (0, 0)))
    grid = (b // _BB,)
    return pl.pallas_call(
        _ln_body,
        grid=grid,
        in_specs=[
            pl.BlockSpec((_BB, s, h2), lambda i: (i, 0, 0)),
            pl.BlockSpec((_BB, s), lambda i: (i, 0)),
            pl.BlockSpec((s, h), lambda i: (0, 0)),
            pl.BlockSpec((8, h), lambda i: (0, 0)),
            pl.BlockSpec((8, h), lambda i: (0, 0)),
            pl.BlockSpec((8, h), lambda i: (0, 0)),
        ],
        out_specs=pl.BlockSpec((_BB, s, h), lambda i: (i, 0, 0)),
        out_shape=jax.ShapeDtypeStruct((b, s, h), jnp.float32),
    )(wemb, token_type_ids, pos_s, ttab, gamma8, beta8)


def kernel(input_ids, token_type_ids, word_table, pos_table, type_table, gamma, beta):
    b, s = input_ids.shape
    h = word_table.shape[1]
    table_pad = jnp.pad(word_table, ((0, 0), (0, h)))
    idx_flat = input_ids.reshape(-1)
    wemb = _sc_gather(table_pad, idx_flat).reshape(b, s, 2 * h)
    pos_s = pos_table[:s]
    return _tc_layernorm(wemb, token_type_ids, pos_s, type_table, gamma, beta)


# speedup vs baseline: 3.0308x; 1.4440x over previous
import functools

import jax
import jax.numpy as jnp
from jax.experimental import pallas as pl
from jax.experimental.pallas import tpu as pltpu
from jax.experimental.pallas import tpu_sc as plsc

_EPS = 1e-12
_GATHER_W = 128
_BB = 8


def _sc_gather(table_pad, idx_flat):
    n = idx_flat.shape[0]
    h2 = table_pad.shape[1]
    mesh = plsc.VectorSubcoreMesh(core_axis_name="c", subcore_axis_name="s")
    idx2 = idx_flat.reshape(1, n)

    @functools.partial(
        pl.kernel,
        out_type=jax.ShapeDtypeStruct((n, h2), jnp.float32),
        mesh=mesh,
        compiler_params=pltpu.CompilerParams(use_tc_tiling_on_sc=False),
    )
    def gather_kernel(table_hbm, i_hbm, o_hbm):
        def body(i_vmem, o_vmem):
            pltpu.sync_copy(table_hbm.at[i_vmem.at[0]], o_vmem)

        pltpu.emit_pipeline(
            body,
            grid=(n // _GATHER_W,),
            in_specs=[pl.BlockSpec((1, _GATHER_W), lambda i: (0, i))],
            out_specs=[pl.BlockSpec((_GATHER_W, h2), lambda i: (i, 0))],
            core_axis_name=("c", "s"),
            dimension_semantics=(pltpu.PARALLEL,),
        )(i_hbm, o_hbm)

    return gather_kernel(table_pad, idx2)


def _tr_body(in_ref, out_ref):
    h = in_ref.shape[0]
    out_ref[:, :h] = in_ref[...].T


def _tc_transpose_pad(table_t):
    h, v = table_t.shape
    c = 8192
    return pl.pallas_call(
        _tr_body,
        grid=((v + c - 1) // c,),
        in_specs=[pl.BlockSpec((h, c), lambda i: (0, i))],
        out_specs=pl.BlockSpec((c, 2 * h), lambda i: (i, 0)),
        out_shape=jax.ShapeDtypeStruct((v, 2 * h), jnp.float32),
    )(table_t)


def _ln_body(wemb_ref, tt_ref, pos_ref, ttab_ref, gamma_ref, beta_ref, out_ref):
    h = out_ref.shape[2]
    emb = wemb_ref[:, :, :h]
    tt = tt_ref[...].astype(jnp.float32)
    pos = pos_ref[...]
    ttab = ttab_ref[...]
    t0 = ttab[0:1, :]
    dt = ttab[1:2, :] - t0
    type_emb = t0[None] + tt[..., None] * dt[None]
    emb = emb + pos[None] + type_emb
    mean = jnp.mean(emb, axis=-1, keepdims=True)
    cen = emb - mean
    var = jnp.mean(cen * cen, axis=-1, keepdims=True)
    normed = cen * jax.lax.rsqrt(var + _EPS)
    gamma = gamma_ref[...][0]
    beta = beta_ref[...][0]
    out_ref[...] = normed * gamma + beta


def _tc_layernorm(wemb, token_type_ids, pos_s, type_table, gamma, beta):
    b, s, h2 = wemb.shape
    h = h2 // 2
    ttab = jnp.pad(type_table, ((0, 6), (0, 0)))
    gamma8 = jnp.pad(gamma.reshape(1, h), ((0, 7), (0, 0)))
    beta8 = jnp.pad(beta.reshape(1, h), ((0, 7), (0, 0)))
    grid = (b // _BB,)
    return pl.pallas_call(
        _ln_body,
        grid=grid,
        in_specs=[
            pl.BlockSpec((_BB, s, h2), lambda i: (i, 0, 0)),
            pl.BlockSpec((_BB, s), lambda i: (i, 0)),
            pl.BlockSpec((s, h), lambda i: (0, 0)),
            pl.BlockSpec((8, h), lambda i: (0, 0)),
            pl.BlockSpec((8, h), lambda i: (0, 0)),
            pl.BlockSpec((8, h), lambda i: (0, 0)),
        ],
        out_specs=pl.BlockSpec((_BB, s, h), lambda i: (i, 0, 0)),
        out_shape=jax.ShapeDtypeStruct((b, s, h), jnp.float32),
    )(wemb, token_type_ids, pos_s, ttab, gamma8, beta8)


def kernel(input_ids, token_type_ids, word_table, pos_table, type_table, gamma, beta):
    b, s = input_ids.shape
    h = word_table.shape[1]
    table_pad = _tc_transpose_pad(jnp.swapaxes(word_table, 0, 1))
    idx_flat = input_ids.reshape(-1)
    wemb = _sc_gather(table_pad, idx_flat).reshape(b, s, 2 * h)
    pos_s = pos_table[:s]
    return _tc_layernorm(wemb, token_type_ids, pos_s, type_table, gamma, beta)


# speedup vs baseline: 3.0779x; 1.0155x over previous
import functools

import jax
import jax.numpy as jnp
from jax.experimental import pallas as pl
from jax.experimental.pallas import tpu as pltpu
from jax.experimental.pallas import tpu_sc as plsc

_EPS = 1e-12
_GATHER_W = 128
_BB = 8


def _sc_gather(table_pad, idx_flat):
    n = idx_flat.shape[0]
    h2 = table_pad.shape[1]
    mesh = plsc.VectorSubcoreMesh(core_axis_name="c", subcore_axis_name="s")
    idx2 = idx_flat.reshape(1, n)

    @functools.partial(
        pl.kernel,
        out_type=jax.ShapeDtypeStruct((n, h2), jnp.float32),
        mesh=mesh,
        compiler_params=pltpu.CompilerParams(use_tc_tiling_on_sc=False),
    )
    def gather_kernel(table_hbm, i_hbm, o_hbm):
        def body(i_vmem, o_vmem):
            pltpu.sync_copy(table_hbm.at[i_vmem.at[0]], o_vmem)

        pltpu.emit_pipeline(
            body,
            grid=(n // _GATHER_W,),
            in_specs=[pl.BlockSpec((1, _GATHER_W), lambda i: (0, i))],
            out_specs=[pl.BlockSpec((_GATHER_W, h2), lambda i: (i, 0))],
            core_axis_name=("c", "s"),
            dimension_semantics=(pltpu.PARALLEL,),
        )(i_hbm, o_hbm)

    return gather_kernel(table_pad, idx2)


def _tr_body(in_ref, out_ref):
    h = in_ref.shape[0]
    out_ref[:, :h] = in_ref[...].T


def _tc_transpose_pad(table_t):
    h, v = table_t.shape
    c = 16384
    return pl.pallas_call(
        _tr_body,
        grid=((v + c - 1) // c,),
        in_specs=[pl.BlockSpec((h, c), lambda i: (0, i))],
        out_specs=pl.BlockSpec((c, 2 * h), lambda i: (i, 0)),
        out_shape=jax.ShapeDtypeStruct((v, 2 * h), jnp.float32),
    )(table_t)


def _ln_body(wemb_ref, tt_ref, pos_ref, ttab_ref, gamma_ref, beta_ref, out_ref):
    h = out_ref.shape[2]
    emb = wemb_ref[:, :, :h]
    tt = tt_ref[...].astype(jnp.float32)
    pos = pos_ref[...]
    ttab = ttab_ref[...]
    t0 = ttab[0:1, :]
    dt = ttab[1:2, :] - t0
    type_emb = t0[None] + tt[..., None] * dt[None]
    emb = emb + pos[None] + type_emb
    mean = jnp.mean(emb, axis=-1, keepdims=True)
    cen = emb - mean
    var = jnp.mean(cen * cen, axis=-1, keepdims=True)
    normed = cen * jax.lax.rsqrt(var + _EPS)
    gamma = gamma_ref[...][0]
    beta = beta_ref[...][0]
    out_ref[...] = normed * gamma + beta


def _tc_layernorm(wemb, token_type_ids, pos_s, type_table, gamma, beta):
    b, s, h2 = wemb.shape
    h = h2 // 2
    ttab = jnp.pad(type_table, ((0, 6), (0, 0)))
    gamma8 = jnp.pad(gamma.reshape(1, h), ((0, 7), (0, 0)))
    beta8 = jnp.pad(beta.reshape(1, h), ((0, 7), (0, 0)))
    grid = (b // _BB,)
    return pl.pallas_call(
        _ln_body,
        grid=grid,
        in_specs=[
            pl.BlockSpec((_BB, s, h2), lambda i: (i, 0, 0)),
            pl.BlockSpec((_BB, s), lambda i: (i, 0)),
            pl.BlockSpec((s, h), lambda i: (0, 0)),
            pl.BlockSpec((8, h), lambda i: (0, 0)),
            pl.BlockSpec((8, h), lambda i: (0, 0)),
            pl.BlockSpec((8, h), lambda i: (0, 0)),
        ],
        out_specs=pl.BlockSpec((_BB, s, h), lambda i: (i, 0, 0)),
        out_shape=jax.ShapeDtypeStruct((b, s, h), jnp.float32),
    )(wemb, token_type_ids, pos_s, ttab, gamma8, beta8)


def kernel(input_ids, token_type_ids, word_table, pos_table, type_table, gamma, beta):
    b, s = input_ids.shape
    h = word_table.shape[1]
    table_pad = _tc_transpose_pad(jnp.swapaxes(word_table, 0, 1))
    pos_s = pos_table[:s]
    bc = b // 2
    outs = []
    for k in range(2):
        ids_k = jax.lax.slice_in_dim(input_ids, k * bc, (k + 1) * bc, axis=0)
        tt_k = jax.lax.slice_in_dim(token_type_ids, k * bc, (k + 1) * bc, axis=0)
        wemb_k = _sc_gather(table_pad, ids_k.reshape(-1)).reshape(bc, s, 2 * h)
        outs.append(_tc_layernorm(wemb_k, tt_k, pos_s, type_table, gamma, beta))
    return jnp.concatenate(outs, axis=0)


# speedup vs baseline: 3.5874x; 1.1655x over previous
import functools

import jax
import jax.numpy as jnp
from jax.experimental import pallas as pl
from jax.experimental.pallas import tpu as pltpu
from jax.experimental.pallas import tpu_sc as plsc

_EPS = 1e-12
_GATHER_W = 128
_BBL = 128
_SC = 40


def _sc_gather(table_pad, idx_flat):
    n = idx_flat.shape[0]
    h2 = table_pad.shape[1]
    mesh = plsc.VectorSubcoreMesh(core_axis_name="c", subcore_axis_name="s")
    idx2 = idx_flat.reshape(1, n)

    @functools.partial(
        pl.kernel,
        out_type=jax.ShapeDtypeStruct((n, h2), jnp.float32),
        mesh=mesh,
        compiler_params=pltpu.CompilerParams(use_tc_tiling_on_sc=False),
    )
    def gather_kernel(table_hbm, i_hbm, o_hbm):
        def body(i_vmem, o_vmem):
            pltpu.sync_copy(table_hbm.at[i_vmem.at[0]], o_vmem)

        pltpu.emit_pipeline(
            body,
            grid=(n // _GATHER_W,),
            in_specs=[pl.BlockSpec((1, _GATHER_W), lambda i: (0, i))],
            out_specs=[pl.BlockSpec((_GATHER_W, h2), lambda i: (i, 0))],
            core_axis_name=("c", "s"),
            dimension_semantics=(pltpu.PARALLEL,),
        )(i_hbm, o_hbm)

    return gather_kernel(table_pad, idx2)


def _tr_body(in_ref, out_ref):
    h = in_ref.shape[0]
    out_ref[:, :h] = in_ref[...].T


def _tc_transpose_pad(table_t):
    h, v = table_t.shape
    c = 16384
    return pl.pallas_call(
        _tr_body,
        grid=((v + c - 1) // c,),
        in_specs=[pl.BlockSpec((h, c), lambda i: (0, i))],
        out_specs=pl.BlockSpec((c, 2 * h), lambda i: (i, 0)),
        out_shape=jax.ShapeDtypeStruct((v, 2 * h), jnp.float32),
    )(table_t)


def _ln_body(wemb_ref, ttt_ref, pos_ref, ttab_ref, gamma_ref, beta_ref, out_ref):
    h = out_ref.shape[1]
    x = wemb_ref[:, :, :h]
    xt = jnp.transpose(x, (1, 2, 0))
    tt = ttt_ref[...].astype(jnp.float32)
    pos = pos_ref[...][:, :, None]
    ttab = ttab_ref[...]
    t0 = ttab[0:1, :][:, :, None]
    dt = ttab[1:2, :][:, :, None] - t0
    emb = xt + pos + t0 + tt[:, None, :] * dt
    mean = jnp.mean(emb, axis=1, keepdims=True)
    cen = emb - mean
    var = jnp.mean(cen * cen, axis=1, keepdims=True)
    normed = cen * jax.lax.rsqrt(var + _EPS)
    gamma = gamma_ref[0:1, :][:, :, None]
    beta = beta_ref[0:1, :][:, :, None]
    out_ref[...] = normed * gamma + beta


def _tc_layernorm(wemb, token_type_ids, pos_s, type_table, gamma, beta):
    b, s, h2 = wemb.shape
    h = h2 // 2
    tt_t = jnp.swapaxes(token_type_ids, 0, 1)
    ttab = jnp.pad(type_table, ((0, 6), (0, 0)))
    gamma8 = jnp.pad(gamma.reshape(1, h), ((0, 7), (0, 0)))
    beta8 = jnp.pad(beta.reshape(1, h), ((0, 7), (0, 0)))
    grid = (b // _BBL, s // _SC)
    out_t = pl.pallas_call(
        _ln_body,
        grid=grid,
        in_specs=[
            pl.BlockSpec((_BBL, _SC, h2), lambda i, j: (i, j, 0)),
            pl.BlockSpec((_SC, _BBL), lambda i, j: (j, i)),
            pl.BlockSpec((_SC, h), lambda i, j: (j, 0)),
            pl.BlockSpec((8, h), lambda i, j: (0, 0)),
            pl.BlockSpec((8, h), lambda i, j: (0, 0)),
            pl.BlockSpec((8, h), lambda i, j: (0, 0)),
        ],
        out_specs=pl.BlockSpec((_SC, h, _BBL), lambda i, j: (j, 0, i)),
        out_shape=jax.ShapeDtypeStruct((s, h, b), jnp.float32),
        compiler_params=pltpu.CompilerParams(vmem_limit_bytes=50 * 2**20),
    )(wemb, tt_t, pos_s, ttab, gamma8, beta8)
    return jnp.transpose(out_t, (2, 0, 1))


def kernel(input_ids, token_type_ids, word_table, pos_table, type_table, gamma, beta):
    b, s = input_ids.shape
    h = word_table.shape[1]
    table_pad = _tc_transpose_pad(jnp.swapaxes(word_table, 0, 1))
    pos_s = pos_table[:s]
    wemb = _sc_gather(table_pad, input_ids.reshape(-1)).reshape(b, s, 2 * h)
    return _tc_layernorm(wemb, token_type_ids, pos_s, type_table, gamma, beta)


# speedup vs baseline: 3.7055x; 1.0329x over previous
import functools

import jax
import jax.numpy as jnp
from jax.experimental import pallas as pl
from jax.experimental.pallas import tpu as pltpu
from jax.experimental.pallas import tpu_sc as plsc

_EPS = 1e-12
_GATHER_W = 128
_BBL = 128
_SC = 40


def _sc_gather(table_pad, idx_flat):
    n = idx_flat.shape[0]
    h2 = table_pad.shape[1]
    mesh = plsc.VectorSubcoreMesh(core_axis_name="c", subcore_axis_name="s")
    idx2 = idx_flat.reshape(1, n)

    sub = 2
    w = sub * _GATHER_W

    @functools.partial(
        pl.kernel,
        out_type=jax.ShapeDtypeStruct((n, h2), jnp.float32),
        mesh=mesh,
        scratch_types=[pltpu.SemaphoreType.DMA],
        compiler_params=pltpu.CompilerParams(use_tc_tiling_on_sc=False),
    )
    def gather_kernel(table_hbm, i_hbm, o_hbm, sem):
        def body(i_vmem, o_vmem):
            cps = []
            for t in range(sub):
                cps.append(pltpu.async_copy(
                    table_hbm.at[i_vmem.at[0, pl.ds(t * _GATHER_W, _GATHER_W)]],
                    o_vmem.at[pl.ds(t * _GATHER_W, _GATHER_W), :],
                    sem,
                ))
            for cp in cps:
                cp.wait()

        pltpu.emit_pipeline(
            body,
            grid=(n // w,),
            in_specs=[pl.BlockSpec((1, w), lambda i: (0, i))],
            out_specs=[pl.BlockSpec((w, h2), lambda i: (i, 0))],
            core_axis_name=("c", "s"),
            dimension_semantics=(pltpu.PARALLEL,),
        )(i_hbm, o_hbm)

    return gather_kernel(table_pad, idx2)


def _tr_body(in_ref, out_ref):
    h = in_ref.shape[0]
    out_ref[:, :h] = in_ref[...].T


def _tc_transpose_pad(table_t):
    h, v = table_t.shape
    c = 16384
    return pl.pallas_call(
        _tr_body,
        grid=((v + c - 1) // c,),
        in_specs=[pl.BlockSpec((h, c), lambda i: (0, i))],
        out_specs=pl.BlockSpec((c, 2 * h), lambda i: (i, 0)),
        out_shape=jax.ShapeDtypeStruct((v, 2 * h), jnp.float32),
    )(table_t)


def _ln_body(wemb_ref, ttt_ref, pos_ref, ttab_ref, gamma_ref, beta_ref, out_ref):
    h = out_ref.shape[1]
    x = wemb_ref[:, :, :h]
    xt = jnp.transpose(x, (1, 2, 0))
    tt = ttt_ref[...].astype(jnp.float32)
    pos = pos_ref[...][:, :, None]
    ttab = ttab_ref[...]
    t0 = ttab[0:1, :][:, :, None]
    dt = ttab[1:2, :][:, :, None] - t0
    emb = xt + pos + t0 + tt[:, None, :] * dt
    mean = jnp.mean(emb, axis=1, keepdims=True)
    cen = emb - mean
    var = jnp.mean(cen * cen, axis=1, keepdims=True)
    normed = cen * jax.lax.rsqrt(var + _EPS)
    gamma = gamma_ref[0:1, :][:, :, None]
    beta = beta_ref[0:1, :][:, :, None]
    out_ref[...] = normed * gamma + beta


def _tc_layernorm(wemb, token_type_ids, pos_s, type_table, gamma, beta):
    b, s, h2 = wemb.shape
    h = h2 // 2
    tt_t = jnp.swapaxes(token_type_ids, 0, 1)
    ttab = jnp.pad(type_table, ((0, 6), (0, 0)))
    gamma8 = jnp.pad(gamma.reshape(1, h), ((0, 7), (0, 0)))
    beta8 = jnp.pad(beta.reshape(1, h), ((0, 7), (0, 0)))
    grid = (b // _BBL, s // _SC)
    out_t = pl.pallas_call(
        _ln_body,
        grid=grid,
        in_specs=[
            pl.BlockSpec((_BBL, _SC, h2), lambda i, j: (i, j, 0)),
            pl.BlockSpec((_SC, _BBL), lambda i, j: (j, i)),
            pl.BlockSpec((_SC, h), lambda i, j: (j, 0)),
            pl.BlockSpec((8, h), lambda i, j: (0, 0)),
            pl.BlockSpec((8, h), lambda i, j: (0, 0)),
            pl.BlockSpec((8, h), lambda i, j: (0, 0)),
        ],
        out_specs=pl.BlockSpec((_SC, h, _BBL), lambda i, j: (j, 0, i)),
        out_shape=jax.ShapeDtypeStruct((s, h, b), jnp.float32),
        compiler_params=pltpu.CompilerParams(vmem_limit_bytes=50 * 2**20),
    )(wemb, tt_t, pos_s, ttab, gamma8, beta8)
    return jnp.transpose(out_t, (2, 0, 1))


def kernel(input_ids, token_type_ids, word_table, pos_table, type_table, gamma, beta):
    b, s = input_ids.shape
    h = word_table.shape[1]
    table_pad = _tc_transpose_pad(jnp.swapaxes(word_table, 0, 1))
    pos_s = pos_table[:s]
    wemb = _sc_gather(table_pad, input_ids.reshape(-1)).reshape(b, s, 2 * h)
    return _tc_layernorm(wemb, token_type_ids, pos_s, type_table, gamma, beta)
